# Initial kernel scaffold; baseline (speedup 1.0000x reference)
import jax
import jax.numpy as jnp
from jax.experimental import pallas as pl


def kernel(x, gate_w, w_gate, w_up, w_down):
    raise NotImplementedError("write your pallas kernel here")


# speedup vs baseline: 1.5533x; 1.5533x over previous
import functools

import jax
import jax.numpy as jnp
from jax.experimental import pallas as pl
from jax.experimental.pallas import tpu as pltpu

TOP_K = 2
BM = 128


def _router_kernel(x_ref, gw_ref, i0_ref, i1_ref, w0_ref, w1_ref, psum_ref,
                   acc_ref):
    g = pl.program_id(0)
    logits = jax.lax.dot_general(
        x_ref[...].astype(jnp.bfloat16), gw_ref[...].astype(jnp.bfloat16),
        (((1,), (0,)), ((), ())),
        preferred_element_type=jnp.float32)
    m1 = jnp.max(logits, axis=-1, keepdims=True)
    i1 = jnp.argmax(logits, axis=-1).astype(jnp.int32)[:, None]
    lane = jax.lax.broadcasted_iota(jnp.int32, logits.shape, 1)
    masked = jnp.where(lane == i1, -jnp.inf, logits)
    m2 = jnp.max(masked, axis=-1, keepdims=True)
    i2 = jnp.argmax(masked, axis=-1).astype(jnp.int32)[:, None]
    e2 = jnp.exp(m2 - m1)
    i0_ref[...] = i1
    i1_ref[...] = i2
    w0_ref[...] = 1.0 / (1.0 + e2)
    w1_ref[...] = e2 / (1.0 + e2)
    p = jnp.exp(logits - m1)
    p = p / jnp.sum(p, axis=-1, keepdims=True)

    @pl.when(g == 0)
    def _():
        acc_ref[...] = jnp.zeros_like(acc_ref)

    acc_ref[...] += jnp.sum(p, axis=0, keepdims=True)

    @pl.when(g == pl.num_programs(0) - 1)
    def _():
        psum_ref[...] = acc_ref[...]


def _ffn_kernel(sp_ref, xb_ref, tok_ref, wcol_ref, wg_hbm, wu_hbm, wd_hbm,
                y_ref, wg_v, wu_v, wd_v, sems, *, nb, t):
    g = pl.program_id(0)
    e = sp_ref[g]
    e_prev = sp_ref[jnp.maximum(g - 1, 0)]
    nvalid = sp_ref[nb]

    @pl.when((g == 0) | (e != e_prev))
    def _():
        c1 = pltpu.make_async_copy(wg_hbm.at[e], wg_v, sems.at[0])
        c2 = pltpu.make_async_copy(wu_hbm.at[e], wu_v, sems.at[1])
        c3 = pltpu.make_async_copy(wd_hbm.at[e], wd_v, sems.at[2])
        c1.start()
        c2.start()
        c3.start()
        c1.wait()
        c2.wait()
        c3.wait()

    @pl.when(g < nvalid)
    def _():
        tok = tok_ref[...]
        col = jax.lax.broadcasted_iota(jnp.int32, (BM, t), 1)
        oh = jnp.where(col == tok, 1.0, 0.0).astype(jnp.bfloat16)
        xs = jax.lax.dot_general(
            oh, xb_ref[...], (((1,), (0,)), ((), ())),
            preferred_element_type=jnp.float32).astype(jnp.bfloat16)
        g1 = jax.lax.dot_general(xs, wg_v[...], (((1,), (0,)), ((), ())),
                                 preferred_element_type=jnp.float32)
        g2 = jax.lax.dot_general(xs, wu_v[...], (((1,), (0,)), ((), ())),
                                 preferred_element_type=jnp.float32)
        h = (g1 * jax.nn.sigmoid(g1) * g2).astype(jnp.bfloat16)
        y = jax.lax.dot_general(h, wd_v[...], (((1,), (0,)), ((), ())),
                                preferred_element_type=jnp.float32)
        y_ref[...] = (y * wcol_ref[...]).astype(jnp.bfloat16)


def _combine_kernel(p0_ref, p1_ref, y_ref, o_ref, *, a_pad):
    p0 = p0_ref[...]
    p1 = p1_ref[...]
    bmc = p0.shape[0]
    col = jax.lax.broadcasted_iota(jnp.int32, (bmc, a_pad), 1)
    oh = ((col == p0) | (col == p1)).astype(jnp.bfloat16)
    o_ref[...] = jax.lax.dot_general(
        oh, y_ref[...], (((1,), (0,)), ((), ())),
        preferred_element_type=jnp.float32)


def kernel(x, gate_w, w_gate, w_up, w_down):
    b, s, d = x.shape
    e_num = gate_w.shape[1]
    f = w_gate.shape[2]
    t = b * s
    a = t * TOP_K
    a_pad = a + e_num * BM
    nb = a_pad // BM

    xf = x.reshape(t, d)
    xb = xf.astype(jnp.bfloat16)
    wg = w_gate.astype(jnp.bfloat16)
    wu = w_up.astype(jnp.bfloat16)
    wd = w_down.astype(jnp.bfloat16)

    bmr = min(512, t)
    i0, i1, w0, w1, psum = pl.pallas_call(
        _router_kernel,
        grid=(t // bmr,),
        in_specs=[
            pl.BlockSpec((bmr, d), lambda g: (g, 0)),
            pl.BlockSpec((d, e_num), lambda g: (0, 0)),
        ],
        out_specs=[
            pl.BlockSpec((bmr, 1), lambda g: (g, 0)),
            pl.BlockSpec((bmr, 1), lambda g: (g, 0)),
            pl.BlockSpec((bmr, 1), lambda g: (g, 0)),
            pl.BlockSpec((bmr, 1), lambda g: (g, 0)),
            pl.BlockSpec((1, e_num), lambda g: (0, 0)),
        ],
        out_shape=[
            jax.ShapeDtypeStruct((t, 1), jnp.int32),
            jax.ShapeDtypeStruct((t, 1), jnp.int32),
            jax.ShapeDtypeStruct((t, 1), jnp.float32),
            jax.ShapeDtypeStruct((t, 1), jnp.float32),
            jax.ShapeDtypeStruct((1, e_num), jnp.float32),
        ],
        scratch_shapes=[pltpu.VMEM((1, e_num), jnp.float32)],
        compiler_params=pltpu.CompilerParams(
            dimension_semantics=("arbitrary",)),
    )(xf, gate_w)

    idx = jnp.concatenate([i0, i1], axis=1)
    rw = jnp.concatenate([w0, w1], axis=1)
    e_a = idx.reshape(-1)
    onehot = (e_a[:, None] == jnp.arange(e_num, dtype=jnp.int32)[None, :])
    cum = jnp.cumsum(onehot.astype(jnp.int32), axis=0)
    rank = jnp.take_along_axis(cum, e_a[:, None], axis=1)[:, 0] - 1
    counts = cum[-1]
    padded = ((counts + BM - 1) // BM) * BM
    ends = jnp.cumsum(padded)
    offsets = ends - padded
    pos = (offsets[e_a] + rank).astype(jnp.int32)
    tok = (jnp.arange(a, dtype=jnp.int32) // TOP_K)
    tok_of_pos = jnp.zeros((a_pad,), jnp.int32).at[pos].set(tok)
    w_of_pos = jnp.zeros((a_pad,), jnp.float32).at[pos].set(rw.reshape(-1))
    nvalid = (ends[-1] // BM).astype(jnp.int32)
    blk_start = jnp.arange(nb, dtype=jnp.int32) * BM
    block_expert = jnp.searchsorted(ends, blk_start, side='right')
    block_expert = jnp.minimum(block_expert, e_num - 1).astype(jnp.int32)
    sp = jnp.concatenate([block_expert, nvalid[None]])

    grid_spec = pltpu.PrefetchScalarGridSpec(
        num_scalar_prefetch=1,
        grid=(nb,),
        in_specs=[
            pl.BlockSpec((t, d), lambda g, sp_: (0, 0)),
            pl.BlockSpec((BM, 1), lambda g, sp_: (g, 0)),
            pl.BlockSpec((BM, 1), lambda g, sp_: (g, 0)),
            pl.BlockSpec(memory_space=pl.ANY),
            pl.BlockSpec(memory_space=pl.ANY),
            pl.BlockSpec(memory_space=pl.ANY),
        ],
        out_specs=pl.BlockSpec((BM, d), lambda g, sp_: (g, 0)),
        scratch_shapes=[
            pltpu.VMEM((d, f), jnp.bfloat16),
            pltpu.VMEM((d, f), jnp.bfloat16),
            pltpu.VMEM((f, d), jnp.bfloat16),
            pltpu.SemaphoreType.DMA((3,)),
        ],
    )
    y = pl.pallas_call(
        functools.partial(_ffn_kernel, nb=nb, t=t),
        grid_spec=grid_spec,
        out_shape=jax.ShapeDtypeStruct((a_pad, d), jnp.bfloat16),
        compiler_params=pltpu.CompilerParams(
            dimension_semantics=("arbitrary",)),
    )(sp, xb, tok_of_pos[:, None], w_of_pos[:, None], wg, wu, wd)

    bmc = min(128, t)
    p0 = pos[0::TOP_K][:, None]
    p1 = pos[1::TOP_K][:, None]
    out = pl.pallas_call(
        functools.partial(_combine_kernel, a_pad=a_pad),
        grid=(t // bmc,),
        in_specs=[
            pl.BlockSpec((bmc, 1), lambda g: (g, 0)),
            pl.BlockSpec((bmc, 1), lambda g: (g, 0)),
            pl.BlockSpec((a_pad, d), lambda g: (0, 0)),
        ],
        out_specs=pl.BlockSpec((bmc, d), lambda g: (g, 0)),
        out_shape=jax.ShapeDtypeStruct((t, d), jnp.float32),
    )(p0, p1, y)

    output = out.reshape(b, s, d)
    probs_mean = psum[0] / t
    frac = counts.astype(jnp.float32) / a
    aux = jnp.sum(probs_mean * frac) * e_num
    return output, aux
